# Initial kernel scaffold; baseline (speedup 1.0000x reference)
import jax
import jax.numpy as jnp
from jax.experimental import pallas as pl


def kernel(x, pos_edge_index, neg_edge_index, Wp0, bp0, Wn0, bn0, Wp1, bp1, Wn1, bn1):
    raise NotImplementedError("write your pallas kernel here")


# speedup vs baseline: 12.9625x; 12.9625x over previous
import functools

import jax
import jax.numpy as jnp
from jax import lax
from jax.experimental import pallas as pl
from jax.experimental.pallas import tpu as pltpu
from jax.experimental.pallas import tpu_sc as plsc

N = 10000
E = 320000
N_PAD = 10240
N_TILES = 16
RPT = N_PAD // N_TILES
EPT = E // N_TILES
CH = 80
NCH = EPT // CH
RB = 80
NRB = RPT // RB

_mesh = plsc.VectorSubcoreMesh(core_axis_name="c", subcore_axis_name="s")
_sc_params = pltpu.CompilerParams(use_tc_tiling_on_sc=False)


@functools.partial(
    pl.kernel,
    out_type=[jax.ShapeDtypeStruct((N_PAD, 16), jnp.float32),
              jax.ShapeDtypeStruct((N_PAD, 16), jnp.float32)],
    mesh=_mesh,
    scratch_types=[pltpu.VMEM_SHARED((N_PAD, 16), jnp.float32),
                   pltpu.VMEM((CH,), jnp.int32),
                   pltpu.VMEM((RB, 16), jnp.float32)],
    compiler_params=_sc_params,
)
def _sc_degrees(pdst_hbm, ndst_hbm, cntp_hbm, cntn_hbm, acc_sh, didx_v, buf_v):
    c = lax.axis_index("c")
    s = lax.axis_index("s")
    r0 = s * RPT

    def run(dst_hbm, out_hbm):
        @pl.loop(0, RB)
        def _(i):
            buf_v[i, :] = jnp.zeros((16,), jnp.float32)

        @pl.loop(0, NRB)
        def _(j):
            pltpu.sync_copy(buf_v, acc_sh.at[pl.ds(r0 + j * RB, RB)])

        plsc.subcore_barrier()

        @pl.loop(0, RB)
        def _(i):
            buf_v[i, :] = jnp.ones((16,), jnp.float32)

        base = s * EPT

        @pl.loop(0, NCH)
        def _(g):
            pltpu.sync_copy(dst_hbm.at[pl.ds(base + g * CH, CH)], didx_v)
            pltpu.sync_copy(buf_v, acc_sh.at[didx_v], add=True)

        plsc.subcore_barrier()

        @pl.loop(0, NRB)
        def _(j):
            pltpu.sync_copy(acc_sh.at[pl.ds(r0 + j * RB, RB)], buf_v)
            pltpu.sync_copy(buf_v, out_hbm.at[pl.ds(r0 + j * RB, RB)])

    @pl.when(c == 0)
    def _():
        run(pdst_hbm, cntp_hbm)

    @pl.when(c == 1)
    def _():
        run(ndst_hbm, cntn_hbm)


def _make_msgpass(D):
    @functools.partial(
        pl.kernel,
        out_type=[jax.ShapeDtypeStruct((N_PAD, D), jnp.float32),
                  jax.ShapeDtypeStruct((N_PAD, D), jnp.float32)],
        mesh=_mesh,
        scratch_types=[pltpu.VMEM_SHARED((N_PAD, D), jnp.float32),
                       pltpu.VMEM((CH,), jnp.int32),
                       pltpu.VMEM((CH,), jnp.int32),
                       pltpu.VMEM((CH, D), jnp.float32)],
        compiler_params=_sc_params,
    )
    def msgpass(tabp_hbm, tabn_hbm, psrc_hbm, pdst_hbm, nsrc_hbm, ndst_hbm,
                outp_hbm, outn_hbm, acc_sh, sidx_v, didx_v, rows_v):
        c = lax.axis_index("c")
        s = lax.axis_index("s")
        r0 = s * RPT

        def run(tab_hbm, src_hbm, dst_hbm, out_hbm):
            @pl.loop(0, NRB)
            def _(j):
                pltpu.sync_copy(tab_hbm.at[pl.ds(r0 + j * RB, RB)], rows_v)
                pltpu.sync_copy(rows_v, acc_sh.at[pl.ds(r0 + j * RB, RB)])

            plsc.subcore_barrier()

            base = s * EPT

            @pl.loop(0, NCH)
            def _(g):
                off = base + g * CH
                pltpu.sync_copy(src_hbm.at[pl.ds(off, CH)], sidx_v)
                pltpu.sync_copy(dst_hbm.at[pl.ds(off, CH)], didx_v)
                pltpu.sync_copy(tab_hbm.at[sidx_v], rows_v)
                pltpu.sync_copy(rows_v, acc_sh.at[didx_v], add=True)

            plsc.subcore_barrier()

            @pl.loop(0, NRB)
            def _(j):
                pltpu.sync_copy(acc_sh.at[pl.ds(r0 + j * RB, RB)], rows_v)
                pltpu.sync_copy(rows_v, out_hbm.at[pl.ds(r0 + j * RB, RB)])

        @pl.when(c == 0)
        def _():
            run(tabp_hbm, psrc_hbm, pdst_hbm, outp_hbm)

        @pl.when(c == 1)
        def _():
            run(tabn_hbm, nsrc_hbm, ndst_hbm, outn_hbm)

    return msgpass


_sc_msgpass_128 = _make_msgpass(128)
_sc_msgpass_64 = _make_msgpass(64)


BLK = 1280
GRID = N_PAD // BLK


def _dinv(cnt_blk):
    return lax.rsqrt(cnt_blk[:, 0:1] + 1.0)


def _tc_a_body(x_ref, cntp_ref, cntn_ref, wp_ref, wn_ref, bp_ref, bn_ref,
               tp_ref, tn_ref):
    dp = _dinv(cntp_ref[...])
    dn = _dinv(cntn_ref[...])
    x = x_ref[...]
    tp_ref[...] = (jnp.dot(x, wp_ref[...], preferred_element_type=jnp.float32)
                   + bp_ref[...]) * dp
    tn_ref[...] = (jnp.dot(x, wn_ref[...], preferred_element_type=jnp.float32)
                   + bn_ref[...]) * dn


_tc_a = pl.pallas_call(
    _tc_a_body,
    grid=(GRID,),
    in_specs=[pl.BlockSpec((BLK, 128), lambda i: (i, 0)),
              pl.BlockSpec((BLK, 16), lambda i: (i, 0)),
              pl.BlockSpec((BLK, 16), lambda i: (i, 0)),
              pl.BlockSpec((128, 128), lambda i: (0, 0)),
              pl.BlockSpec((128, 128), lambda i: (0, 0)),
              pl.BlockSpec((1, 128), lambda i: (0, 0)),
              pl.BlockSpec((1, 128), lambda i: (0, 0))],
    out_specs=[pl.BlockSpec((BLK, 128), lambda i: (i, 0)),
               pl.BlockSpec((BLK, 128), lambda i: (i, 0))],
    out_shape=[jax.ShapeDtypeStruct((N_PAD, 128), jnp.float32),
               jax.ShapeDtypeStruct((N_PAD, 128), jnp.float32)],
)


def _tc_b_body(accp_ref, accn_ref, cntp_ref, cntn_ref, wp_ref, wn_ref,
               bp_ref, bn_ref, tp_ref, tn_ref):
    dp = _dinv(cntp_ref[...])
    dn = _dinv(cntn_ref[...])
    h = jnp.maximum(accp_ref[...] * dp - accn_ref[...] * dn, 0.0)
    tp_ref[...] = (jnp.dot(h, wp_ref[...], preferred_element_type=jnp.float32)
                   + bp_ref[...]) * dp
    tn_ref[...] = (jnp.dot(h, wn_ref[...], preferred_element_type=jnp.float32)
                   + bn_ref[...]) * dn


_tc_b = pl.pallas_call(
    _tc_b_body,
    grid=(GRID,),
    in_specs=[pl.BlockSpec((BLK, 128), lambda i: (i, 0)),
              pl.BlockSpec((BLK, 128), lambda i: (i, 0)),
              pl.BlockSpec((BLK, 16), lambda i: (i, 0)),
              pl.BlockSpec((BLK, 16), lambda i: (i, 0)),
              pl.BlockSpec((128, 64), lambda i: (0, 0)),
              pl.BlockSpec((128, 64), lambda i: (0, 0)),
              pl.BlockSpec((1, 64), lambda i: (0, 0)),
              pl.BlockSpec((1, 64), lambda i: (0, 0))],
    out_specs=[pl.BlockSpec((BLK, 64), lambda i: (i, 0)),
               pl.BlockSpec((BLK, 64), lambda i: (i, 0))],
    out_shape=[jax.ShapeDtypeStruct((N_PAD, 64), jnp.float32),
               jax.ShapeDtypeStruct((N_PAD, 64), jnp.float32)],
)


def _tc_c_body(accp_ref, accn_ref, cntp_ref, cntn_ref, o_ref):
    dp = _dinv(cntp_ref[...])
    dn = _dinv(cntn_ref[...])
    o_ref[...] = accp_ref[...] * dp - accn_ref[...] * dn


_tc_c = pl.pallas_call(
    _tc_c_body,
    grid=(GRID,),
    in_specs=[pl.BlockSpec((BLK, 64), lambda i: (i, 0)),
              pl.BlockSpec((BLK, 64), lambda i: (i, 0)),
              pl.BlockSpec((BLK, 16), lambda i: (i, 0)),
              pl.BlockSpec((BLK, 16), lambda i: (i, 0))],
    out_specs=pl.BlockSpec((BLK, 64), lambda i: (i, 0)),
    out_shape=jax.ShapeDtypeStruct((N_PAD, 64), jnp.float32),
)


def kernel(x, pos_edge_index, neg_edge_index,
           Wp0, bp0, Wn0, bn0, Wp1, bp1, Wn1, bn1):
    psrc = pos_edge_index[0].astype(jnp.int32)
    pdst = pos_edge_index[1].astype(jnp.int32)
    nsrc = neg_edge_index[0].astype(jnp.int32)
    ndst = neg_edge_index[1].astype(jnp.int32)

    x_pad = jnp.pad(x, ((0, N_PAD - N), (0, 0)))

    cntp, cntn = _sc_degrees(pdst, ndst)

    tab0p, tab0n = _tc_a(x_pad, cntp, cntn, Wp0, Wn0,
                         bp0.reshape(1, 128), bn0.reshape(1, 128))
    acc0p, acc0n = _sc_msgpass_128(tab0p, tab0n, psrc, pdst, nsrc, ndst)

    tab1p, tab1n = _tc_b(acc0p, acc0n, cntp, cntn, Wp1, Wn1,
                         bp1.reshape(1, 64), bn1.reshape(1, 64))
    acc1p, acc1n = _sc_msgpass_64(tab1p, tab1n, psrc, pdst, nsrc, ndst)

    out_pad = _tc_c(acc1p, acc1n, cntp, cntn)
    return out_pad[:N]
